# Initial kernel scaffold; baseline (speedup 1.0000x reference)
import jax
import jax.numpy as jnp
from jax.experimental import pallas as pl


def kernel(x, edge_idx, W1, b1, W2, b2):
    raise NotImplementedError("write your pallas kernel here")


# speedup vs baseline: 14.9140x; 14.9140x over previous
import functools

import jax
import jax.numpy as jnp
from jax import lax
from jax.experimental import pallas as pl
from jax.experimental.pallas import tpu as pltpu
from jax.experimental.pallas import tpu_sc as plsc

N = 10000
E = 320000
D = 128

NC, NS = 2, 16
NW = NC * NS
EPW = E // NW
CH = 128
NFULL = EPW // CH
TAIL = EPW - NFULL * CH
RPT = 632
LAST = N - (NS - 1) * RPT

_sc_mesh = plsc.VectorSubcoreMesh(core_axis_name="c", subcore_axis_name="s")


def _zero_owned(zsrc_hbm, acc, s):
    off = pl.multiple_of(s * RPT, 8)

    @pl.when(s < NS - 1)
    def _full():
        pltpu.sync_copy(zsrc_hbm, acc.at[pl.ds(off, RPT)])

    @pl.when(s == NS - 1)
    def _tail():
        pltpu.sync_copy(zsrc_hbm.at[pl.ds(0, LAST)], acc.at[pl.ds(off, LAST)])


def _writeback_owned(acc, out_hbm, c, s):
    off = pl.multiple_of(s * RPT, 8)

    @pl.when(s < NS - 1)
    def _full():
        pltpu.sync_copy(acc.at[pl.ds(off, RPT)], out_hbm.at[c, pl.ds(off, RPT)])

    @pl.when(s == NS - 1)
    def _tail():
        pltpu.sync_copy(acc.at[pl.ds(off, LAST)], out_hbm.at[c, pl.ds(off, LAST)])


@functools.partial(
    pl.kernel,
    out_type=jax.ShapeDtypeStruct((NC, N, D), jnp.float32),
    mesh=_sc_mesh,
    scratch_types=[
        pltpu.VMEM_SHARED((N, D), jnp.float32),
        pltpu.VMEM((CH, D), jnp.float32),
        pltpu.VMEM((CH,), jnp.int32),
        pltpu.VMEM((TAIL,), jnp.int32),
    ],
)
def _deg_kernel(dst_hbm, ones_hbm, zrow_hbm, out_hbm, acc, ones_v, didx, didx_t):
    c = lax.axis_index("c")
    s = lax.axis_index("s")
    base = (c * NS + s) * EPW
    _zero_owned(zrow_hbm, acc, s)
    pltpu.sync_copy(ones_hbm, ones_v)
    plsc.subcore_barrier()

    def body(k, carry):
        e0 = pl.multiple_of(base + k * CH, 8)
        pltpu.sync_copy(dst_hbm.at[pl.ds(e0, CH)], didx)
        pltpu.sync_copy(ones_v, acc.at[didx], add=True)
        return carry

    lax.fori_loop(0, NFULL, body, 0)
    e0 = pl.multiple_of(base + NFULL * CH, 8)
    pltpu.sync_copy(dst_hbm.at[pl.ds(e0, TAIL)], didx_t)
    pltpu.sync_copy(ones_v.at[pl.ds(0, TAIL)], acc.at[didx_t], add=True)
    plsc.subcore_barrier()
    _writeback_owned(acc, out_hbm, c, s)


@functools.partial(
    pl.kernel,
    out_type=jax.ShapeDtypeStruct((NC, N, D), jnp.float32),
    mesh=_sc_mesh,
    scratch_types=[
        pltpu.VMEM_SHARED((N, D), jnp.float32),
        pltpu.VMEM((CH,), jnp.int32),
        pltpu.VMEM((CH,), jnp.int32),
        pltpu.VMEM((CH, D), jnp.float32),
        pltpu.VMEM((TAIL,), jnp.int32),
        pltpu.VMEM((TAIL,), jnp.int32),
        pltpu.VMEM((TAIL, D), jnp.float32),
        pltpu.SemaphoreType.DMA,
    ],
)
def _agg_kernel(h_hbm, src_hbm, dst_hbm, zrow_hbm, out_hbm,
                acc, sidx, didx, rows, sidx_t, didx_t, rows_t, sem):
    c = lax.axis_index("c")
    s = lax.axis_index("s")
    base = (c * NS + s) * EPW
    _zero_owned(zrow_hbm, acc, s)
    plsc.subcore_barrier()

    def body(k, carry):
        e0 = pl.multiple_of(base + k * CH, 8)
        pltpu.sync_copy(src_hbm.at[pl.ds(e0, CH)], sidx)
        pltpu.sync_copy(dst_hbm.at[pl.ds(e0, CH)], didx)
        pltpu.async_copy(h_hbm.at[sidx], rows, sem).wait()
        pltpu.sync_copy(rows, acc.at[didx], add=True)
        return carry

    lax.fori_loop(0, NFULL, body, 0)
    e0 = pl.multiple_of(base + NFULL * CH, 8)
    pltpu.sync_copy(src_hbm.at[pl.ds(e0, TAIL)], sidx_t)
    pltpu.sync_copy(dst_hbm.at[pl.ds(e0, TAIL)], didx_t)
    pltpu.async_copy(h_hbm.at[sidx_t], rows_t, sem).wait()
    pltpu.sync_copy(rows_t, acc.at[didx_t], add=True)
    plsc.subcore_barrier()
    _writeback_owned(acc, out_hbm, c, s)


BN = 1000
G = N // BN


def _dinv_block(p_ref):
    deg = p_ref[0][:, 0:1] + p_ref[1][:, 0:1] + 1.0
    return lax.rsqrt(deg)


def _tc1_body(x_ref, w_ref, p_ref, o_ref):
    h = jnp.dot(x_ref[...], w_ref[...], preferred_element_type=jnp.float32)
    o_ref[...] = h * _dinv_block(p_ref)


def _tc2_body(q_ref, hp_ref, p_ref, b_ref, w_ref, o_ref):
    dinv = _dinv_block(p_ref)
    agg = q_ref[0] + q_ref[1] + hp_ref[...]
    z = jnp.maximum(agg * dinv + b_ref[...], 0.0)
    o_ref[...] = jnp.dot(z, w_ref[...], preferred_element_type=jnp.float32) * dinv


def _tc3_body(q_ref, hp_ref, p_ref, b_ref, o_ref):
    dinv = _dinv_block(p_ref)
    agg = q_ref[0] + q_ref[1] + hp_ref[...]
    o_ref[...] = agg * dinv + b_ref[...]


_row_spec = pl.BlockSpec((BN, D), lambda i: (i, 0))
_p_spec = pl.BlockSpec((NC, BN, D), lambda i: (0, i, 0))
_q_spec = pl.BlockSpec((NC, BN, D), lambda i: (0, i, 0))
_w_spec = pl.BlockSpec((D, D), lambda i: (0, 0))
_b_spec = pl.BlockSpec((1, D), lambda i: (0, 0))
_out_sds = jax.ShapeDtypeStruct((N, D), jnp.float32)

_tc1 = pl.pallas_call(
    _tc1_body, grid=(G,),
    in_specs=[_row_spec, _w_spec, _p_spec],
    out_specs=_row_spec, out_shape=_out_sds,
)
_tc2 = pl.pallas_call(
    _tc2_body, grid=(G,),
    in_specs=[_q_spec, _row_spec, _p_spec, _b_spec, _w_spec],
    out_specs=_row_spec, out_shape=_out_sds,
)
_tc3 = pl.pallas_call(
    _tc3_body, grid=(G,),
    in_specs=[_q_spec, _row_spec, _p_spec, _b_spec],
    out_specs=_row_spec, out_shape=_out_sds,
)


def kernel(x, edge_idx, W1, b1, W2, b2):
    src = edge_idx[0]
    dst = edge_idx[1]
    ones_c = jnp.ones((CH, D), jnp.float32)
    zrow = jnp.zeros((RPT, D), jnp.float32)
    b1r = b1.reshape(1, D)
    b2r = b2.reshape(1, D)

    p = _deg_kernel(dst, ones_c, zrow)
    h1p = _tc1(x, W1, p)
    q1 = _agg_kernel(h1p, src, dst, zrow)
    h2p = _tc2(q1, h1p, p, b1r, W2)
    q2 = _agg_kernel(h2p, src, dst, zrow)
    out = _tc3(q2, h2p, p, b2r)
    return out


# speedup vs baseline: 26.7878x; 1.7962x over previous
import functools

import jax
import jax.numpy as jnp
from jax import lax
from jax.experimental import pallas as pl
from jax.experimental.pallas import tpu as pltpu
from jax.experimental.pallas import tpu_sc as plsc

N = 10000
E = 320000
D = 128

NC, NS = 2, 16
NW = NC * NS
CH = 64
NCH = 162
NB = 3
E_PAD = NW * NCH * CH
JROWS = 128
NROWS = N + JROWS
RPT = 632
LAST = N - (NS - 1) * RPT

_sc_mesh = plsc.VectorSubcoreMesh(core_axis_name="c", subcore_axis_name="s")


def _zero_owned(zsrc_hbm, acc, s):
    off = pl.multiple_of(s * RPT, 8)

    @pl.when(s < NS - 1)
    def _full():
        pltpu.sync_copy(zsrc_hbm, acc.at[pl.ds(off, RPT)])

    @pl.when(s == NS - 1)
    def _tail():
        pltpu.sync_copy(zsrc_hbm.at[pl.ds(0, LAST)], acc.at[pl.ds(off, LAST)])


def _writeback_owned(acc, out_hbm, c, s):
    off = pl.multiple_of(s * RPT, 8)

    @pl.when(s < NS - 1)
    def _full():
        pltpu.sync_copy(acc.at[pl.ds(off, RPT)], out_hbm.at[c, pl.ds(off, RPT)])

    @pl.when(s == NS - 1)
    def _tail():
        pltpu.sync_copy(acc.at[pl.ds(off, LAST)], out_hbm.at[c, pl.ds(off, LAST)])


@functools.partial(
    pl.kernel,
    out_type=jax.ShapeDtypeStruct((NC, N, D), jnp.float32),
    mesh=_sc_mesh,
    scratch_types=[
        pltpu.VMEM_SHARED((NROWS, D), jnp.float32),
        pltpu.VMEM((CH, D), jnp.float32),
        pltpu.VMEM((NCH, CH), jnp.int32),
        pltpu.SemaphoreType.DMA,
    ],
)
def _deg_kernel(dst3_hbm, ones_hbm, zrow_hbm, out_hbm, acc, ones_v, didx_v, ssem):
    c = lax.axis_index("c")
    s = lax.axis_index("s")
    wid = c * NS + s
    _zero_owned(zrow_hbm, acc, s)
    pltpu.sync_copy(ones_hbm, ones_v)
    pltpu.sync_copy(dst3_hbm.at[wid], didx_v)
    plsc.subcore_barrier()

    def fire(k, carry):
        pltpu.async_copy(ones_v, acc.at[didx_v.at[k]], ssem, add=True)
        return carry

    lax.fori_loop(0, NCH, fire, 0)

    def drain(k, carry):
        pltpu.make_async_copy(ones_v, acc.at[didx_v.at[0]], ssem).wait()
        return carry

    lax.fori_loop(0, NCH, drain, 0)
    plsc.subcore_barrier()
    _writeback_owned(acc, out_hbm, c, s)


MASK = (1 << 14) - 1


def _unpack_row(pk_v, k, out_small, shift):
    for m in range(CH // 16):
        v = pk_v[k, pl.ds(m * 16, 16)]
        out_small[0, pl.ds(m * 16, 16)] = (v >> shift) & MASK


@functools.partial(
    pl.kernel,
    out_type=jax.ShapeDtypeStruct((NC, N, D), jnp.float32),
    mesh=_sc_mesh,
    scratch_types=[
        pltpu.VMEM_SHARED((NROWS, D), jnp.float32),
        pltpu.VMEM((NCH, CH), jnp.int32),
    ]
    + [pltpu.VMEM((CH, D), jnp.float32)] * NB
    + [pltpu.VMEM((1, CH), jnp.int32)] * NB
    + [pltpu.VMEM((1, CH), jnp.int32)] * NB
    + [pltpu.SemaphoreType.DMA] * (2 * NB),
)
def _agg_kernel(h_hbm, pk3_hbm, zrow_hbm, out_hbm, acc, pk_v, *bufs):
    rows = bufs[:NB]
    sidx = bufs[NB:2 * NB]
    didx = bufs[2 * NB:3 * NB]
    gsem = bufs[3 * NB:4 * NB]
    ssem = bufs[4 * NB:]
    c = lax.axis_index("c")
    s = lax.axis_index("s")
    wid = c * NS + s
    _zero_owned(zrow_hbm, acc, s)
    pltpu.sync_copy(pk3_hbm.at[wid], pk_v)
    plsc.subcore_barrier()

    for b in range(NB - 1):
        _unpack_row(pk_v, b, sidx[b], 0)
        pltpu.async_copy(h_hbm.at[sidx[b].at[0]], rows[b], gsem[b])

    def group(g, carry):
        for b in range(NB):
            k = g * NB + b
            bj = (b + NB - 1) % NB
            pltpu.make_async_copy(h_hbm.at[sidx[b].at[0]], rows[b], gsem[b]).wait()
            _unpack_row(pk_v, k, didx[b], 14)
            pltpu.async_copy(rows[b], acc.at[didx[b].at[0]], ssem[b], add=True)

            @pl.when(k >= 1)
            def _drain_prev():
                pltpu.make_async_copy(rows[bj], acc.at[didx[bj].at[0]],
                                      ssem[bj]).wait()

            @pl.when(k + NB - 1 < NCH)
            def _prefetch():
                _unpack_row(pk_v, k + NB - 1, sidx[bj], 0)
                pltpu.async_copy(h_hbm.at[sidx[bj].at[0]], rows[bj], gsem[bj])

        return carry

    lax.fori_loop(0, NCH // NB, group, 0)
    bl = (NCH - 1) % NB
    pltpu.make_async_copy(rows[bl], acc.at[didx[bl].at[0]], ssem[bl]).wait()
    plsc.subcore_barrier()
    _writeback_owned(acc, out_hbm, c, s)


BN = 1000
G = N // BN


def _dinv_block(p_ref):
    deg = p_ref[0][:, 0:1] + p_ref[1][:, 0:1] + 1.0
    return lax.rsqrt(deg)


def _tc0_body(x_ref, w_ref, o_ref):
    o_ref[...] = jnp.dot(x_ref[...], w_ref[...], preferred_element_type=jnp.float32)


def _tc1_body(h_ref, p_ref, o_ref):
    o_ref[...] = h_ref[...] * _dinv_block(p_ref)


def _tc2_body(q_ref, hp_ref, p_ref, b_ref, w_ref, o_ref):
    dinv = _dinv_block(p_ref)
    agg = q_ref[0] + q_ref[1] + hp_ref[...]
    z = jnp.maximum(agg * dinv + b_ref[...], 0.0)
    o_ref[...] = jnp.dot(z, w_ref[...], preferred_element_type=jnp.float32) * dinv


def _tc3_body(q_ref, hp_ref, p_ref, b_ref, o_ref):
    dinv = _dinv_block(p_ref)
    agg = q_ref[0] + q_ref[1] + hp_ref[...]
    o_ref[...] = agg * dinv + b_ref[...]


_row_spec = pl.BlockSpec((BN, D), lambda i: (i, 0))
_p_spec = pl.BlockSpec((NC, BN, D), lambda i: (0, i, 0))
_q_spec = pl.BlockSpec((NC, BN, D), lambda i: (0, i, 0))
_w_spec = pl.BlockSpec((D, D), lambda i: (0, 0))
_b_spec = pl.BlockSpec((1, D), lambda i: (0, 0))
_out_sds = jax.ShapeDtypeStruct((N, D), jnp.float32)

_tc0 = pl.pallas_call(
    _tc0_body, grid=(G,),
    in_specs=[_row_spec, _w_spec],
    out_specs=_row_spec, out_shape=_out_sds,
)
_tc1 = pl.pallas_call(
    _tc1_body, grid=(G,),
    in_specs=[_row_spec, _p_spec],
    out_specs=_row_spec, out_shape=_out_sds,
)
_tc2 = pl.pallas_call(
    _tc2_body, grid=(G,),
    in_specs=[_q_spec, _row_spec, _p_spec, _b_spec, _w_spec],
    out_specs=_row_spec, out_shape=_out_sds,
)
_tc3 = pl.pallas_call(
    _tc3_body, grid=(G,),
    in_specs=[_q_spec, _row_spec, _p_spec, _b_spec],
    out_specs=_row_spec, out_shape=_out_sds,
)


def kernel(x, edge_idx, W1, b1, W2, b2):
    src = edge_idx[0]
    dst = edge_idx[1]
    npad = E_PAD - E
    pad_src = (jnp.arange(npad, dtype=jnp.int32) * 131) % N
    pad_dst = N + jnp.arange(npad, dtype=jnp.int32) % JROWS
    dst3 = jnp.concatenate([dst, pad_dst]).reshape(NW, NCH, CH)
    srcp = jnp.concatenate([src, pad_src])
    pk3 = (srcp | (jnp.concatenate([dst, pad_dst]) << 14)).reshape(NW, NCH, CH)
    ones_c = jnp.ones((CH, D), jnp.float32)
    zrow = jnp.zeros((RPT, D), jnp.float32)
    b1r = b1.reshape(1, D)
    b2r = b2.reshape(1, D)

    h1 = _tc0(x, W1)
    p = _deg_kernel(dst3, ones_c, zrow)
    h1p = _tc1(h1, p)
    q1 = _agg_kernel(h1p, pk3, zrow)
    h2p = _tc2(q1, h1p, p, b1r, W2)
    q2 = _agg_kernel(h2p, pk3, zrow)
    out = _tc3(q2, h2p, p, b2r)
    return out
